# Initial kernel scaffold; baseline (speedup 1.0000x reference)
import jax
import jax.numpy as jnp
from jax.experimental import pallas as pl


def kernel(x, edge_index, W_gcn0, b_gcn0, W_int0, b_int0, W_gcn1, b_gcn1, W_int1, b_int1, W_gcn2, b_gcn2, W_int2, b_int2):
    raise NotImplementedError("write your pallas kernel here")


# speedup vs baseline: 2.2965x; 2.2965x over previous
import functools

import jax
import jax.numpy as jnp
from jax import lax
from jax.experimental import pallas as pl
from jax.experimental.pallas import tpu as pltpu

N = 10000
E = 320000
D = 128
NC = 2
NS = 16
CH = 100
G = 100
NP = 10240
RPT = NP // NS
CB = 128


R = 400
NB = N // R


def _leaky(t):
    return jnp.where(t >= 0, t, 0.2 * t)


def _tc_prescale_body(d, x, dis_o, g_o):
    dd = d[...]
    deg = dd[0, :, 0:1] + dd[1, :, 0:1]
    dis = jnp.where(deg > 0, lax.rsqrt(jnp.maximum(deg, 1e-12)), 0.0)
    dis_o[...] = dis
    g_o[...] = x[...] * dis


_tc_prescale = pl.pallas_call(
    _tc_prescale_body,
    grid=(NB,),
    in_specs=[
        pl.BlockSpec((NC, R, 16), lambda i: (0, i, 0)),
        pl.BlockSpec((R, D), lambda i: (i, 0)),
    ],
    out_specs=(
        pl.BlockSpec((R, 1), lambda i: (i, 0)),
        pl.BlockSpec((R, D), lambda i: (i, 0)),
    ),
    out_shape=(
        jax.ShapeDtypeStruct((N, 1), jnp.float32),
        jax.ShapeDtypeStruct((N, D), jnp.float32),
    ),
)


def _tc_layer_body(p, h, dis, Wg, bg, Wi, bi, hn_o, gn_o, o_o):
    dis_b = dis[...]
    pp = p[...]
    m = (pp[0] + pp[1]) * dis_b
    hc = h[...]
    t1 = jnp.dot(m, Wg[...], preferred_element_type=jnp.float32,
                 precision=lax.Precision.HIGHEST) + bg[...]
    t2 = jnp.dot(hc * m, Wi[...], preferred_element_type=jnp.float32,
                 precision=lax.Precision.HIGHEST) + bi[...]
    hn = _leaky(t1) + _leaky(t2)
    sq = jnp.sum(hn * hn, axis=-1, keepdims=True)
    o_o[...] = hn * lax.rsqrt(jnp.maximum(sq, 1e-12))
    hn_o[...] = hn
    gn_o[...] = hn * dis_b


_tc_layer = pl.pallas_call(
    _tc_layer_body,
    grid=(NB,),
    in_specs=[
        pl.BlockSpec((NC, R, D), lambda i: (0, i, 0)),
        pl.BlockSpec((R, D), lambda i: (i, 0)),
        pl.BlockSpec((R, 1), lambda i: (i, 0)),
        pl.BlockSpec((D, D), lambda i: (0, 0)),
        pl.BlockSpec((D,), lambda i: (0,)),
        pl.BlockSpec((D, D), lambda i: (0, 0)),
        pl.BlockSpec((D,), lambda i: (0,)),
    ],
    out_specs=(
        pl.BlockSpec((R, D), lambda i: (i, 0)),
        pl.BlockSpec((R, D), lambda i: (i, 0)),
        pl.BlockSpec((R, D), lambda i: (i, 0)),
    ),
    out_shape=(
        jax.ShapeDtypeStruct((N, D), jnp.float32),
        jax.ShapeDtypeStruct((N, D), jnp.float32),
        jax.ShapeDtypeStruct((N, D), jnp.float32),
    ),
)


def kernel(x, edge_index, W_gcn0, b_gcn0, W_int0, b_int0,
           W_gcn1, b_gcn1, W_int1, b_int1,
           W_gcn2, b_gcn2, W_int2, b_int2):
    row = edge_index[0]
    col = edge_index[1]
    r0, r1 = row[:E // 2], row[E // 2:]
    c0, c1 = col[:E // 2], col[E // 2:]
    dp0 = jnp.zeros((NP,), jnp.float32).at[r0].add(1.0)
    dp1 = jnp.zeros((NP,), jnp.float32).at[r1].add(1.0)
    d = jnp.stack([jnp.broadcast_to(dp0[:, None], (NP, 16)),
                   jnp.broadcast_to(dp1[:, None], (NP, 16))])
    dis, g = _tc_prescale(d, x)
    params = [(W_gcn0, b_gcn0, W_int0, b_int0),
              (W_gcn1, b_gcn1, W_int1, b_int1),
              (W_gcn2, b_gcn2, W_int2, b_int2)]
    h = x
    outs = []
    for (Wg, bg, Wi, bi) in params:
        p0 = jnp.zeros((NP, D), jnp.float32).at[r0].add(g[c0])
        p1 = jnp.zeros((NP, D), jnp.float32).at[r1].add(g[c1])
        p = jnp.stack([p0, p1])
        h, g, o = _tc_layer(p, h, dis, Wg, bg, Wi, bi)
        outs.append(o)
    return jnp.concatenate([x] + outs, axis=-1)
